# baseline (speedup 1.0000x reference)
import functools

import numpy as np
import jax
import jax.numpy as jnp
from jax import lax
from jax.experimental import pallas as pl
from jax.experimental.pallas import tpu as pltpu
from jax.experimental.pallas import tpu_sc as plsc

VOCAB = 100000
EMBED = 128
MAX_LEN = 512
B, L = 1024, 200

_NUM_CORES = 2
_NUM_SUBCORES = 16
_NW = _NUM_CORES * _NUM_SUBCORES
_LANES = 16
_CH = 64
_RPW = (B * L) // _NW
_CPW = _RPW // _CH
_NBUF = 5
_LOOK = 3
_NGRP = _CPW // _NBUF
_PEROWS = 256


def _sinusoidal_pe(max_len, d_model):
    position = np.arange(max_len, dtype=np.float64)[:, None]
    div_term = np.exp(
        np.arange(0, d_model, 2, dtype=np.float64) * -(np.log(10000.0) / d_model)
    )
    pe = np.zeros((max_len, d_model), dtype=np.float64)
    pe[:, 0::2] = np.sin(position * div_term)
    pe[:, 1::2] = np.cos(position * div_term)
    return pe.astype(np.float32)


_PE = _sinusoidal_pe(MAX_LEN, EMBED)[:L]
_PE2 = np.concatenate([_PE, _PE], axis=0)[:_PEROWS]


def _sc_body(table_hbm, idx_hbm, pe_hbm, out_hbm, idx_v, pe_v, rows, sem_g, sem_s):
    wid = lax.axis_index("s") * _NUM_CORES + lax.axis_index("c")
    base = wid * _RPW

    pltpu.sync_copy(pe_hbm, pe_v)
    pltpu.sync_copy(idx_hbm.at[pl.ds(base, _RPW)], idx_v)

    def gather(c, b):
        return pltpu.make_async_copy(
            table_hbm.at[idx_v.at[pl.ds(c * _CH, _CH)]], rows[b], sem_g[b]
        )

    def store(c, b):
        return pltpu.make_async_copy(
            rows[b], out_hbm.at[pl.ds(base + c * _CH, _CH)], sem_s[b]
        )

    def add_pe(c, b):
        l0 = lax.rem(c * _CH, L) if not isinstance(c, int) else (c * _CH) % L

        @pl.loop(0, _CH)
        def _add(r):
            for d in range(EMBED // _LANES):
                sl = pl.ds(d * _LANES, _LANES)
                plsc.addupdate(rows[b].at[r, sl], pe_v[l0 + r, sl])

    def visit(c, b, first_group, last_group):
        gather(c, b).wait()
        add_pe(c, b)
        store(c, b).start()
        pb = (b + _LOOK) % _NBUF
        if last_group:
            if isinstance(c, int) and c + _LOOK < _CPW:
                store(c - (_NBUF - _LOOK), pb).wait()
                gather(c + _LOOK, pb).start()
        else:
            if not (first_group and b + _LOOK < _NBUF):
                store(c - (_NBUF - _LOOK), pb).wait()
            gather(c + _LOOK, pb).start()

    for b in range(_LOOK):
        gather(b, b).start()

    for b in range(_NBUF):
        visit(b, b, True, False)

    @pl.loop(1, _NGRP - 1)
    def _group(g):
        for b in range(_NBUF):
            visit(g * _NBUF + b, b, False, False)

    for b in range(_NBUF):
        visit((_NGRP - 1) * _NBUF + b, b, False, True)

    for c in range(_CPW - _NBUF, _CPW):
        store(c, c % _NBUF).wait()


@functools.partial(
    pl.kernel,
    out_type=jax.ShapeDtypeStruct((B * L, EMBED), jnp.float32),
    mesh=plsc.VectorSubcoreMesh(core_axis_name="c", subcore_axis_name="s"),
    scratch_types=[
        pltpu.VMEM((_RPW,), jnp.int32),
        pltpu.VMEM((_PEROWS, EMBED), jnp.float32),
        [pltpu.VMEM((_CH, EMBED), jnp.float32) for _ in range(_NBUF)],
        [pltpu.SemaphoreType.DMA for _ in range(_NBUF)],
        [pltpu.SemaphoreType.DMA for _ in range(_NBUF)],
    ],
)
def _sc_embed(table_hbm, idx_hbm, pe_hbm, out_hbm, idx_v, pe_v, rows, sem_g, sem_s):
    _sc_body(table_hbm, idx_hbm, pe_hbm, out_hbm, idx_v, pe_v, rows, sem_g, sem_s)


def kernel(sequence, token_table):
    idx = sequence.reshape(-1).astype(jnp.int32)
    out = _sc_embed(token_table, idx, jnp.asarray(_PE2))
    return out.reshape(B, L, EMBED)


# speedup vs baseline: 2.0082x; 2.0082x over previous
import functools

import numpy as np
import jax
import jax.numpy as jnp
from jax import lax
from jax.experimental import pallas as pl
from jax.experimental.pallas import tpu as pltpu
from jax.experimental.pallas import tpu_sc as plsc

VOCAB = 100000
EMBED = 128
MAX_LEN = 512
B, L = 1024, 200

_NUM_CORES = 2
_NUM_SUBCORES = 16
_NW = _NUM_CORES * _NUM_SUBCORES
_LANES = 16
_CH = 128
_RPW = (B * L) // _NW
_CPW = _RPW // _CH
_NBUF = 4
_LOOK = 2
_SKIP_GATHER = False
_SKIP_ADD = False


def _sinusoidal_pe(max_len, d_model):
    position = np.arange(max_len, dtype=np.float64)[:, None]
    div_term = np.exp(
        np.arange(0, d_model, 2, dtype=np.float64) * -(np.log(10000.0) / d_model)
    )
    pe = np.zeros((max_len, d_model), dtype=np.float64)
    pe[:, 0::2] = np.sin(position * div_term)
    pe[:, 1::2] = np.cos(position * div_term)
    return pe.astype(np.float32)


_PE = _sinusoidal_pe(MAX_LEN, EMBED)[:L]
_PE2 = np.concatenate([_PE, _PE], axis=0)


def _sc_body(table_hbm, idx_hbm, pe_hbm, out_hbm, idx_v, pe_v, rows, sem_g, sem_s):
    wid = lax.axis_index("s") * _NUM_CORES + lax.axis_index("c")
    base = wid * _RPW

    pltpu.sync_copy(pe_hbm, pe_v)
    pltpu.sync_copy(idx_hbm.at[pl.ds(base, _RPW)], idx_v)

    def gather(c):
        b = c % _NBUF
        return pltpu.make_async_copy(
            table_hbm.at[idx_v.at[pl.ds(c * _CH, _CH)]], rows[b], sem_g[b]
        )

    def store(c):
        b = c % _NBUF
        return pltpu.make_async_copy(
            rows[b], out_hbm.at[pl.ds(base + c * _CH, _CH)], sem_s[b]
        )

    def wait_store(c):
        store(c).wait()

    if not _SKIP_GATHER:
        for c in range(_LOOK):
            gather(c).start()

    for c in range(_CPW):
        if not _SKIP_GATHER:
            gather(c).wait()
        b = c % _NBUF
        l0 = (c * _CH) % L

        p = c + _LOOK
        if p < _CPW:
            if p >= _NBUF:
                wait_store(p - _NBUF)
            if not _SKIP_GATHER:
                gather(p).start()

        if not _SKIP_ADD:
            @pl.loop(0, _CH, unroll=4)
            def _add(r, b=b, l0=l0):
                for d in range(EMBED // _LANES):
                    sl = pl.ds(d * _LANES, _LANES)
                    plsc.addupdate(rows[b].at[r, sl], pe_v[l0 + r, sl])
        store(c).start()

    for c in range(max(0, _CPW - _NBUF), _CPW):
        wait_store(c)


@functools.partial(
    pl.kernel,
    out_type=jax.ShapeDtypeStruct((B * L, EMBED), jnp.float32),
    mesh=plsc.VectorSubcoreMesh(core_axis_name="c", subcore_axis_name="s"),
    scratch_types=[
        pltpu.VMEM((_RPW,), jnp.int32),
        pltpu.VMEM((2 * L, EMBED), jnp.float32),
        [pltpu.VMEM((_CH, EMBED), jnp.float32) for _ in range(_NBUF)],
        [pltpu.SemaphoreType.DMA for _ in range(_NBUF)],
        [pltpu.SemaphoreType.DMA for _ in range(_NBUF)],
    ],
)
def _sc_embed(table_hbm, idx_hbm, pe_hbm, out_hbm, idx_v, pe_v, rows, sem_g, sem_s):
    _sc_body(table_hbm, idx_hbm, pe_hbm, out_hbm, idx_v, pe_v, rows, sem_g, sem_s)


def kernel(sequence, token_table):
    idx = sequence.reshape(-1).astype(jnp.int32)
    out = _sc_embed(token_table, idx, jnp.asarray(_PE2))
    return out.reshape(B, L, EMBED)


# speedup vs baseline: 2.0329x; 1.0123x over previous
import functools

import numpy as np
import jax
import jax.numpy as jnp
from jax import lax
from jax.experimental import pallas as pl
from jax.experimental.pallas import tpu as pltpu
from jax.experimental.pallas import tpu_sc as plsc

VOCAB = 100000
EMBED = 128
MAX_LEN = 512
B, L = 1024, 200

_NUM_CORES = 2
_NUM_SUBCORES = 16
_NW = _NUM_CORES * _NUM_SUBCORES
_LANES = 16
_CH = 128
_RPW = (B * L) // _NW
_CPW = _RPW // _CH
_NBUF = 5
_LOOK = 2
_SKIP_GATHER = False
_SKIP_ADD = False


def _sinusoidal_pe(max_len, d_model):
    position = np.arange(max_len, dtype=np.float64)[:, None]
    div_term = np.exp(
        np.arange(0, d_model, 2, dtype=np.float64) * -(np.log(10000.0) / d_model)
    )
    pe = np.zeros((max_len, d_model), dtype=np.float64)
    pe[:, 0::2] = np.sin(position * div_term)
    pe[:, 1::2] = np.cos(position * div_term)
    return pe.astype(np.float32)


_PE = _sinusoidal_pe(MAX_LEN, EMBED)[:L]
_PE2 = np.concatenate([_PE, _PE], axis=0)[:320]


def _sc_body(table_hbm, idx_hbm, pe_hbm, out_hbm, idx_v, pe_v, rows, sem_g, sem_s):
    wid = lax.axis_index("s") * _NUM_CORES + lax.axis_index("c")
    base = wid * _RPW

    pltpu.sync_copy(pe_hbm, pe_v)
    pltpu.sync_copy(idx_hbm.at[pl.ds(base, _RPW)], idx_v)

    def gather(c):
        b = c % _NBUF
        return pltpu.make_async_copy(
            table_hbm.at[idx_v.at[pl.ds(c * _CH, _CH)]], rows[b], sem_g[b]
        )

    def store(c):
        b = c % _NBUF
        return pltpu.make_async_copy(
            rows[b], out_hbm.at[pl.ds(base + c * _CH, _CH)], sem_s[b]
        )

    def wait_store(c):
        store(c).wait()

    if not _SKIP_GATHER:
        for c in range(_LOOK):
            gather(c).start()

    for c in range(_CPW):
        if not _SKIP_GATHER:
            gather(c).wait()
        b = c % _NBUF
        l0 = (c * _CH) % L

        p = c + _LOOK
        if p < _CPW:
            if p >= _NBUF:
                wait_store(p - _NBUF)
            if not _SKIP_GATHER:
                gather(p).start()

        if not _SKIP_ADD:
            @pl.loop(0, _CH, unroll=4)
            def _add(r, b=b, l0=l0):
                for d in range(EMBED // _LANES):
                    sl = pl.ds(d * _LANES, _LANES)
                    plsc.addupdate(rows[b].at[r, sl], pe_v[l0 + r, sl])
        store(c).start()

    for c in range(max(0, _CPW - _NBUF), _CPW):
        wait_store(c)


@functools.partial(
    pl.kernel,
    out_type=jax.ShapeDtypeStruct((B * L, EMBED), jnp.float32),
    mesh=plsc.VectorSubcoreMesh(core_axis_name="c", subcore_axis_name="s"),
    scratch_types=[
        pltpu.VMEM((_RPW,), jnp.int32),
        pltpu.VMEM((320, EMBED), jnp.float32),
        [pltpu.VMEM((_CH, EMBED), jnp.float32) for _ in range(_NBUF)],
        [pltpu.SemaphoreType.DMA for _ in range(_NBUF)],
        [pltpu.SemaphoreType.DMA for _ in range(_NBUF)],
    ],
)
def _sc_embed(table_hbm, idx_hbm, pe_hbm, out_hbm, idx_v, pe_v, rows, sem_g, sem_s):
    _sc_body(table_hbm, idx_hbm, pe_hbm, out_hbm, idx_v, pe_v, rows, sem_g, sem_s)


def kernel(sequence, token_table):
    idx = sequence.reshape(-1).astype(jnp.int32)
    out = _sc_embed(token_table, idx, jnp.asarray(_PE2))
    return out.reshape(B, L, EMBED)


# speedup vs baseline: 2.0586x; 1.0127x over previous
import functools

import numpy as np
import jax
import jax.numpy as jnp
from jax import lax
from jax.experimental import pallas as pl
from jax.experimental.pallas import tpu as pltpu
from jax.experimental.pallas import tpu_sc as plsc

VOCAB = 100000
EMBED = 128
MAX_LEN = 512
B, L = 1024, 200

_NUM_CORES = 2
_NUM_SUBCORES = 16
_NW = _NUM_CORES * _NUM_SUBCORES
_LANES = 16
_CH = 128
_RPW = (B * L) // _NW
_CPW = _RPW // _CH
_NBUF = 5
_LOOK = 2
_SKIP_GATHER = False
_SKIP_ADD = False


def _sinusoidal_pe(max_len, d_model):
    position = np.arange(max_len, dtype=np.float64)[:, None]
    div_term = np.exp(
        np.arange(0, d_model, 2, dtype=np.float64) * -(np.log(10000.0) / d_model)
    )
    pe = np.zeros((max_len, d_model), dtype=np.float64)
    pe[:, 0::2] = np.sin(position * div_term)
    pe[:, 1::2] = np.cos(position * div_term)
    return pe.astype(np.float32)


_PE = _sinusoidal_pe(MAX_LEN, EMBED)[:L]
_PE2 = np.concatenate([_PE, _PE], axis=0)[:320]


def _sc_body(table_hbm, idx_hbm, pe_hbm, out_hbm, idx_v, pe_v, rows, sem_g, sem_s):
    wid = lax.axis_index("s") * _NUM_CORES + lax.axis_index("c")
    base = wid * _RPW

    pe_copy = pltpu.make_async_copy(pe_hbm, pe_v, sem_s[0])
    pe_copy.start()
    pltpu.sync_copy(idx_hbm.at[pl.ds(base, _RPW)], idx_v)

    def gather(c):
        b = c % _NBUF
        return pltpu.make_async_copy(
            table_hbm.at[idx_v.at[pl.ds(c * _CH, _CH)]], rows[b], sem_g[b]
        )

    def store(c):
        b = c % _NBUF
        return pltpu.make_async_copy(
            rows[b], out_hbm.at[pl.ds(base + c * _CH, _CH)], sem_s[b]
        )

    def wait_store(c):
        store(c).wait()

    if not _SKIP_GATHER:
        for c in range(_LOOK):
            gather(c).start()
    pe_copy.wait()

    for c in range(_CPW):
        if not _SKIP_GATHER:
            gather(c).wait()
        b = c % _NBUF
        l0 = (c * _CH) % L

        p = c + _LOOK
        if p < _CPW:
            if p >= _NBUF:
                wait_store(p - _NBUF)
            if not _SKIP_GATHER:
                gather(p).start()

        if not _SKIP_ADD:
            @pl.loop(0, _CH, unroll=4)
            def _add(r, b=b, l0=l0):
                for d in range(EMBED // _LANES):
                    sl = pl.ds(d * _LANES, _LANES)
                    plsc.addupdate(rows[b].at[r, sl], pe_v[l0 + r, sl])
        store(c).start()

    for c in range(max(0, _CPW - _NBUF), _CPW):
        wait_store(c)


@functools.partial(
    pl.kernel,
    out_type=jax.ShapeDtypeStruct((B * L, EMBED), jnp.float32),
    mesh=plsc.VectorSubcoreMesh(core_axis_name="c", subcore_axis_name="s"),
    scratch_types=[
        pltpu.VMEM((_RPW,), jnp.int32),
        pltpu.VMEM((320, EMBED), jnp.float32),
        [pltpu.VMEM((_CH, EMBED), jnp.float32) for _ in range(_NBUF)],
        [pltpu.SemaphoreType.DMA for _ in range(_NBUF)],
        [pltpu.SemaphoreType.DMA for _ in range(_NBUF)],
    ],
)
def _sc_embed(table_hbm, idx_hbm, pe_hbm, out_hbm, idx_v, pe_v, rows, sem_g, sem_s):
    _sc_body(table_hbm, idx_hbm, pe_hbm, out_hbm, idx_v, pe_v, rows, sem_g, sem_s)


def kernel(sequence, token_table):
    idx = sequence.reshape(-1).astype(jnp.int32)
    out = _sc_embed(token_table, idx, jnp.asarray(_PE2))
    return out.reshape(B, L, EMBED)


# speedup vs baseline: 2.1091x; 1.0245x over previous
import functools

import numpy as np
import jax
import jax.numpy as jnp
from jax import lax
from jax.experimental import pallas as pl
from jax.experimental.pallas import tpu as pltpu
from jax.experimental.pallas import tpu_sc as plsc

VOCAB = 100000
EMBED = 128
MAX_LEN = 512
B, L = 1024, 200

_NUM_CORES = 2
_NUM_SUBCORES = 16
_NW = _NUM_CORES * _NUM_SUBCORES
_LANES = 16
_CH = L
_RPW = (B * L) // _NW
_CPW = _RPW // _CH
_NBUF = 4
_LOOK = 2
_G0 = 128
_G1 = _CH - _G0


def _sinusoidal_pe(max_len, d_model):
    position = np.arange(max_len, dtype=np.float64)[:, None]
    div_term = np.exp(
        np.arange(0, d_model, 2, dtype=np.float64) * -(np.log(10000.0) / d_model)
    )
    pe = np.zeros((max_len, d_model), dtype=np.float64)
    pe[:, 0::2] = np.sin(position * div_term)
    pe[:, 1::2] = np.cos(position * div_term)
    return pe.astype(np.float32)


_PE = _sinusoidal_pe(MAX_LEN, EMBED)[:L]


def _sc_body(table_hbm, idx_hbm, pe_hbm, out_hbm, idx_v, pe_v, rows, sem_i, sem_g, sem_s):
    wid = lax.axis_index("s") * _NUM_CORES + lax.axis_index("c")
    base = wid * _RPW

    pe_copy = pltpu.make_async_copy(pe_hbm, pe_v, sem_s[0])
    pe_copy.start()

    def idx_copy(c):
        b = c % _NBUF
        return pltpu.make_async_copy(
            idx_hbm.at[pl.ds(base + c * _CH, _CH)],
            idx_v.at[pl.ds(b * _CH, _CH)],
            sem_i[b],
        )

    def gathers(c):
        b = c % _NBUF
        return (
            pltpu.make_async_copy(
                table_hbm.at[idx_v.at[pl.ds(b * _CH, _G0)]],
                rows[b].at[pl.ds(0, _G0), :],
                sem_g[b],
            ),
            pltpu.make_async_copy(
                table_hbm.at[idx_v.at[pl.ds(b * _CH + _G0, _G1)]],
                rows[b].at[pl.ds(_G0, _G1), :],
                sem_g[b],
            ),
        )

    def store(c):
        b = c % _NBUF
        return pltpu.make_async_copy(
            rows[b], out_hbm.at[pl.ds(base + c * _CH, _CH)], sem_s[b]
        )

    for c in range(min(_LOOK + 1, _CPW)):
        idx_copy(c).start()
    for c in range(_LOOK):
        idx_copy(c).wait()
        g0, g1 = gathers(c)
        g0.start()
        g1.start()
    pe_copy.wait()

    for c in range(_CPW):
        g0, g1 = gathers(c)
        g0.wait()
        g1.wait()
        b = c % _NBUF

        p = c + _LOOK
        if p < _CPW:
            if p >= _NBUF:
                store(p - _NBUF).wait()
            idx_copy(p).wait()
            n0, n1 = gathers(p)
            n0.start()
            n1.start()
            if p + 1 < _CPW:
                idx_copy(p + 1).start()

        @pl.loop(0, _CH, unroll=4)
        def _add(r, b=b):
            for d in range(EMBED // _LANES):
                sl = pl.ds(d * _LANES, _LANES)
                plsc.addupdate(rows[b].at[r, sl], pe_v[r, sl])

        store(c).start()

    for c in range(max(0, _CPW - _NBUF), _CPW):
        store(c).wait()


@functools.partial(
    pl.kernel,
    out_type=jax.ShapeDtypeStruct((B * L, EMBED), jnp.float32),
    mesh=plsc.VectorSubcoreMesh(core_axis_name="c", subcore_axis_name="s"),
    scratch_types=[
        pltpu.VMEM((_NBUF * _CH,), jnp.int32),
        pltpu.VMEM((L, EMBED), jnp.float32),
        [pltpu.VMEM((_CH, EMBED), jnp.float32) for _ in range(_NBUF)],
        [pltpu.SemaphoreType.DMA for _ in range(_NBUF)],
        [pltpu.SemaphoreType.DMA for _ in range(_NBUF)],
        [pltpu.SemaphoreType.DMA for _ in range(_NBUF)],
    ],
)
def _sc_embed(table_hbm, idx_hbm, pe_hbm, out_hbm, idx_v, pe_v, rows, sem_i, sem_g, sem_s):
    _sc_body(table_hbm, idx_hbm, pe_hbm, out_hbm, idx_v, pe_v, rows, sem_i, sem_g, sem_s)


def kernel(sequence, token_table):
    idx = sequence.reshape(-1).astype(jnp.int32)
    out = _sc_embed(token_table, idx, jnp.asarray(_PE))
    return out.reshape(B, L, EMBED)
